# Initial kernel scaffold; baseline (speedup 1.0000x reference)
import jax
import jax.numpy as jnp
from jax.experimental import pallas as pl


def kernel(x0, edge_index, W1, b1, Wr1, br1, W2, b2, Wr2, br2, Wl, bl):
    raise NotImplementedError("write your pallas kernel here")


# speedup vs baseline: 14.9336x; 14.9336x over previous
import functools

import jax
import jax.numpy as jnp
from jax import lax
from jax.experimental import pallas as pl
from jax.experimental.pallas import tpu as pltpu
from jax.experimental.pallas import tpu_sc as plsc

N = 10000
D = 128
C = 64
NPAD = 10240
NDUM = NPAD - N
NC = 2
NS = 16
NW = NC * NS
L = 16
E = 320000
K = 64
BLK = 160
EPT = BLK * K
EPAD = NW * EPT
RPT = NPAD // NS
CHB = 8
NCH = BLK // CHB
CHD = 2048
NCHD = EPT // CHD
CHDR = CHD // 128
NR = NPAD // 128

@functools.cache
def _build_deg_kernel():
    mesh = plsc.VectorSubcoreMesh(
        core_axis_name="c", subcore_axis_name="s",
        num_cores=NC, num_subcores=NS)
    return functools.partial(
        pl.kernel,
        out_type=jax.ShapeDtypeStruct((NW, NR, 128), jnp.float32),
        mesh=mesh,
        scratch_types=[
            pltpu.VMEM((2, CHDR, 128), jnp.int32),
            pltpu.VMEM((NR, 128), jnp.float32),
            pltpu.SemaphoreType.DMA,
        ],
        compiler_params=pltpu.CompilerParams(needs_layout_passes=False),
    )(_deg_body)


def _deg_body(rows_hbm, out_hbm, stage, degv, semd):
    c = lax.axis_index("c")
    s = lax.axis_index("s")
    wid = c * NS + s
    zeros16 = jnp.zeros((L,), jnp.float32)
    ones16 = jnp.ones((L,), jnp.float32)

    def zbody(r, carry):
        for l in range(128 // L):
            degv[r, pl.ds(l * L, L)] = zeros16
        return carry

    lax.fori_loop(0, NR, zbody, 0, unroll=2)
    pltpu.async_copy(
        rows_hbm.at[wid, pl.ds(0, CHDR)], stage.at[0], semd).wait()

    def chunk(q, carry):
        a = lax.rem(q, 2)
        na = 1 - a
        nxt = (q + 1) * CHDR
        pltpu.async_copy(rows_hbm.at[wid, pl.ds(nxt, CHDR)], stage.at[na], semd)

        def body(i, carry2):
            r = lax.shift_right_logical(i, 3)
            l = lax.rem(i, 8)
            idx = stage[a, r, pl.ds(l * L, L)]
            hi = lax.shift_right_logical(idx, 7)
            lo = lax.rem(idx, 128)
            plsc.addupdate_scatter(degv, [hi, lo], ones16)
            return carry2

        lax.fori_loop(0, CHD // L, body, 0, unroll=4)
        pltpu.make_async_copy(
            rows_hbm.at[wid, pl.ds(nxt, CHDR)], stage.at[na], semd).wait()
        return carry

    lax.fori_loop(0, NCHD, chunk, 0)
    pltpu.sync_copy(degv, out_hbm.at[wid])


@functools.cache
def _build_spmm_kernel():
    mesh = plsc.VectorSubcoreMesh(
        core_axis_name="c", subcore_axis_name="s",
        num_cores=NC, num_subcores=NS)
    return functools.partial(
        pl.kernel,
        out_type=jax.ShapeDtypeStruct((NC, NPAD, D), jnp.float32),
        mesh=mesh,
        scratch_types=[
            pltpu.VMEM((2, CHB, K), jnp.int32),
            pltpu.VMEM((2, CHB, K), jnp.int32),
            pltpu.VMEM((2, K, D), jnp.float32),
            pltpu.VMEM_SHARED((NPAD, D), jnp.float32),
            pltpu.SemaphoreType.DMA,
            pltpu.SemaphoreType.DMA,
            pltpu.SemaphoreType.DMA,
            pltpu.SemaphoreType.DMA,
        ],
        compiler_params=pltpu.CompilerParams(needs_layout_passes=False),
    )(_spmm_body)


def _spmm_body(h_hbm, col_hbm, row_hbm, out_hbm, cstage, rstage, buf, aggs,
               semc, semr, sem0, sem1):
    c = lax.axis_index("c")
    s = lax.axis_index("s")
    wid = c * NS + s

    zeros16 = jnp.zeros((L,), jnp.float32)

    def zbody(r, carry):
        for l in range(D // L):
            buf[0, r, pl.ds(l * L, L)] = zeros16
        return carry

    lax.fori_loop(0, K, zbody, 0, unroll=2)
    for j in range(RPT // K):
        pltpu.sync_copy(buf.at[0], aggs.at[pl.ds(s * RPT + j * K, K)])
    plsc.subcore_barrier()

    pltpu.async_copy(
        col_hbm.at[wid, pl.ds(0, CHB)], cstage.at[0], semc).wait()
    pltpu.async_copy(
        row_hbm.at[wid, pl.ds(0, CHB)], rstage.at[0], semr).wait()
    pltpu.async_copy(h_hbm.at[cstage.at[0, 0]], buf.at[0], sem0)

    def chunk_body(q, carry):
        a = lax.rem(q, 2)
        na = 1 - a
        nxt = (q + 1) * CHB
        pltpu.async_copy(
            col_hbm.at[wid, pl.ds(nxt, CHB)], cstage.at[na], semc)
        pltpu.async_copy(
            row_hbm.at[wid, pl.ds(nxt, CHB)], rstage.at[na], semr)
        for b in range(CHB - 1):
            sl = b % 2
            sem = sem0 if sl == 0 else sem1
            nsem = sem1 if sl == 0 else sem0
            pltpu.make_async_copy(
                h_hbm.at[cstage.at[a, b]], buf.at[sl], sem).wait()
            pltpu.async_copy(
                h_hbm.at[cstage.at[a, b + 1]], buf.at[1 - sl], nsem)
            pltpu.sync_copy(buf.at[sl], aggs.at[rstage.at[a, b]], add=True)
        pltpu.make_async_copy(
            col_hbm.at[wid, pl.ds(nxt, CHB)], cstage.at[na], semc).wait()
        pltpu.make_async_copy(
            row_hbm.at[wid, pl.ds(nxt, CHB)], rstage.at[na], semr).wait()
        b = CHB - 1
        pltpu.make_async_copy(
            h_hbm.at[cstage.at[a, b]], buf.at[1], sem1).wait()
        pltpu.async_copy(h_hbm.at[cstage.at[na, 0]], buf.at[0], sem0)
        pltpu.sync_copy(buf.at[1], aggs.at[rstage.at[a, b]], add=True)
        return carry

    lax.fori_loop(0, NCH, chunk_body, 0)
    pltpu.make_async_copy(
        h_hbm.at[cstage.at[0, 0]], buf.at[0], sem0).wait()

    plsc.subcore_barrier()
    pltpu.sync_copy(aggs.at[pl.ds(s * RPT, RPT)],
                    out_hbm.at[c, pl.ds(s * RPT, RPT)])


_BN = 128


def _dinv_rows(degp):
    d = jnp.sum(degp, axis=0)
    dinv = jnp.where(d > 0, lax.rsqrt(d), 0.0)
    rows = jnp.broadcast_to(dinv[None, :], (_BN, _BN))
    eye = (lax.broadcasted_iota(jnp.int32, (_BN, _BN), 0)
           == lax.broadcasted_iota(jnp.int32, (_BN, _BN), 1))
    dm = jnp.where(eye, rows, 0.0)
    ones = jnp.ones((_BN, D), jnp.float32)
    return jnp.dot(dm, ones, preferred_element_type=jnp.float32)


def _l1_body(x_ref, degp_ref, w1t_ref, wr1t_ref, b1_ref, br1_ref,
             h_ref, r_ref, dv_ref):
    dinv_b = _dinv_rows(degp_ref[...].reshape(NW, 128))
    x = x_ref[...]
    t = jnp.dot(x, w1t_ref[...], preferred_element_type=jnp.float32) + b1_ref[...]
    h_ref[...] = dinv_b * t
    r_ref[...] = jnp.dot(x, wr1t_ref[...],
                         preferred_element_type=jnp.float32) + br1_ref[...]
    dv_ref[...] = dinv_b


def _mid_body(aggp_ref, dv_ref, r1_ref, w2t_ref, wr2t_ref, b2_ref, br2_ref,
              x1_ref, h2_ref, r2_ref):
    dinv_b = dv_ref[...]
    a = aggp_ref[0] + aggp_ref[1]
    x1 = jnp.maximum(dinv_b * a + r1_ref[...], 0.0)
    x1_ref[...] = x1
    t = jnp.dot(x1, w2t_ref[...], preferred_element_type=jnp.float32) + b2_ref[...]
    h2_ref[...] = dinv_b * t
    r2_ref[...] = jnp.dot(x1, wr2t_ref[...],
                          preferred_element_type=jnp.float32) + br2_ref[...]


def _fin_body(aggp_ref, dv_ref, r2_ref, x1_ref, wlt_ref, bl_ref, out_ref):
    a = aggp_ref[0] + aggp_ref[1]
    x2 = jnp.maximum(dv_ref[...] * a + r2_ref[...], 0.0)
    xc = jnp.concatenate([x1_ref[...], x2], axis=1)
    t = jnp.dot(xc, wlt_ref[...], preferred_element_type=jnp.float32) + bl_ref[...]
    m = jnp.max(t, axis=1, keepdims=True)
    lse = jnp.log(jnp.sum(jnp.exp(t - m), axis=1, keepdims=True))
    out_ref[...] = t - m - lse


def _row_spec(width):
    return pl.BlockSpec((_BN, width), lambda i: (i, 0))


def _full_spec(shape):
    nd = len(shape)
    return pl.BlockSpec(shape, lambda i, _n=nd: (0,) * _n)


_l1_call = pl.pallas_call(
    _l1_body,
    grid=(NPAD // _BN,),
    in_specs=[
        _row_spec(D),
        pl.BlockSpec((1, NW, 128), lambda i: (i, 0, 0)),
        _full_spec((D, D)), _full_spec((D, D)),
        _full_spec((1, D)), _full_spec((1, D)),
    ],
    out_specs=[_row_spec(D), _row_spec(D), _row_spec(D)],
    out_shape=[jax.ShapeDtypeStruct((NPAD, D), jnp.float32)] * 3,
)

_mid_call = pl.pallas_call(
    _mid_body,
    grid=(NPAD // _BN,),
    in_specs=[
        pl.BlockSpec((NC, _BN, D), lambda i: (0, i, 0)),
        _row_spec(D), _row_spec(D),
        _full_spec((D, D)), _full_spec((D, D)),
        _full_spec((1, D)), _full_spec((1, D)),
    ],
    out_specs=[_row_spec(D), _row_spec(D), _row_spec(D)],
    out_shape=[jax.ShapeDtypeStruct((NPAD, D), jnp.float32)] * 3,
)

_fin_call = pl.pallas_call(
    _fin_body,
    grid=(-(-N // _BN),),
    in_specs=[
        pl.BlockSpec((NC, _BN, D), lambda i: (0, i, 0)),
        _row_spec(D), _row_spec(D), _row_spec(D),
        _full_spec((2 * D, C)), _full_spec((1, C)),
    ],
    out_specs=pl.BlockSpec((_BN, C), lambda i: (i, 0)),
    out_shape=jax.ShapeDtypeStruct((N, C), jnp.float32),
)


def kernel(x0, edge_index, W1, b1, Wr1, br1, W2, b2, Wr2, br2, Wl, bl):
    row = edge_index[0]
    col = edge_index[1]
    npads = EPAD - E
    pad_ids = N + lax.iota(jnp.int32, npads) % NDUM
    rowp = jnp.concatenate([row, pad_ids])
    colp = jnp.concatenate([col, pad_ids])
    dumd = (N + lax.iota(jnp.int32, NW * CHD) % NDUM).reshape(NW, CHD)
    row_flat = jnp.concatenate(
        [rowp.reshape(NW, EPT), dumd], axis=1).reshape(NW, EPT // 128 + CHDR,
                                                       128)
    dumb = (N + lax.iota(jnp.int32, NW * CHB * K) % NDUM).reshape(NW, CHB, K)
    row3 = jnp.concatenate([rowp.reshape(NW, BLK, K), dumb], axis=1)
    col3 = jnp.concatenate([colp.reshape(NW, BLK, K), dumb], axis=1)

    x0p = jnp.pad(x0, ((0, NPAD - N), (0, 0)))
    b1r = b1.reshape(1, D)
    br1r = br1.reshape(1, D)
    b2r = b2.reshape(1, D)
    br2r = br2.reshape(1, D)
    blr = bl.reshape(1, C)

    degp = jnp.transpose(_deg_kernel(row_flat), (1, 0, 2))
    h1p, r1p, dinv_b = _l1_call(x0p, degp, W1.T, Wr1.T, b1r, br1r)
    agg1 = _spmm_kernel(h1p, col3, row3)
    x1, h2p, r2p = _mid_call(agg1, dinv_b, r1p, W2.T, Wr2.T, b2r, br2r)
    agg2 = _spmm_kernel(h2p, col3, row3)
    return _fin_call(agg2, dinv_b, r2p, x1, Wl.T, blr)


def _deg_kernel(rows_hbm):
    return _build_deg_kernel()(rows_hbm)


def _spmm_kernel(h, col3, row3):
    return _build_spmm_kernel()(h, col3, row3)


# speedup vs baseline: 19.4432x; 1.3020x over previous
import functools

import jax
import jax.numpy as jnp
from jax import lax
from jax.experimental import pallas as pl
from jax.experimental.pallas import tpu as pltpu
from jax.experimental.pallas import tpu_sc as plsc

N = 10000
D = 128
C = 64
NPAD = 10240
NDUM = NPAD - N
NC = 2
NS = 16
NW = NC * NS
L = 16
E = 320000
K = 64
BLK = 160
EPT = BLK * K
EPAD = NW * EPT
RPT = NPAD // NS
CHB = 8
NCH = BLK // CHB
NBUF = 4
LA = 2
CHD = 2048
NCHD = EPT // CHD
CHDR = CHD // 128
NR = NPAD // 128

@functools.cache
def _build_deg_kernel():
    mesh = plsc.VectorSubcoreMesh(
        core_axis_name="c", subcore_axis_name="s",
        num_cores=NC, num_subcores=NS)
    return functools.partial(
        pl.kernel,
        out_type=jax.ShapeDtypeStruct((NW, NR, 128), jnp.float32),
        mesh=mesh,
        scratch_types=[
            pltpu.VMEM((2, CHDR, 128), jnp.int32),
            pltpu.VMEM((NR, 128), jnp.float32),
            pltpu.SemaphoreType.DMA,
        ],
        compiler_params=pltpu.CompilerParams(needs_layout_passes=False),
    )(_deg_body)


def _deg_body(rows_hbm, out_hbm, stage, degv, semd):
    c = lax.axis_index("c")
    s = lax.axis_index("s")
    wid = c * NS + s
    zeros16 = jnp.zeros((L,), jnp.float32)
    ones16 = jnp.ones((L,), jnp.float32)

    def zbody(r, carry):
        for l in range(128 // L):
            degv[r, pl.ds(l * L, L)] = zeros16
        return carry

    lax.fori_loop(0, NR, zbody, 0, unroll=2)
    pltpu.async_copy(
        rows_hbm.at[wid, pl.ds(0, CHDR)], stage.at[0], semd).wait()

    def chunk(q, carry):
        a = lax.rem(q, 2)
        na = 1 - a
        nxt = (q + 1) * CHDR
        pltpu.async_copy(rows_hbm.at[wid, pl.ds(nxt, CHDR)], stage.at[na], semd)

        def body(i, carry2):
            r = lax.shift_right_logical(i, 3)
            l = lax.rem(i, 8)
            idx = stage[a, r, pl.ds(l * L, L)]
            hi = lax.shift_right_logical(idx, 7)
            lo = lax.rem(idx, 128)
            plsc.addupdate_scatter(degv, [hi, lo], ones16)
            return carry2

        lax.fori_loop(0, CHD // L, body, 0, unroll=4)
        pltpu.make_async_copy(
            rows_hbm.at[wid, pl.ds(nxt, CHDR)], stage.at[na], semd).wait()
        return carry

    lax.fori_loop(0, NCHD, chunk, 0)
    pltpu.sync_copy(degv, out_hbm.at[wid])


@functools.cache
def _build_spmm_kernel():
    mesh = plsc.VectorSubcoreMesh(
        core_axis_name="c", subcore_axis_name="s",
        num_cores=NC, num_subcores=NS)
    return functools.partial(
        pl.kernel,
        out_type=jax.ShapeDtypeStruct((NC, NPAD, D), jnp.float32),
        mesh=mesh,
        scratch_types=[
            pltpu.VMEM((2, CHB, K), jnp.int32),
            pltpu.VMEM((2, CHB, K), jnp.int32),
            pltpu.VMEM((NBUF, K, D), jnp.float32),
            pltpu.VMEM_SHARED((NPAD, D), jnp.float32),
            pltpu.SemaphoreType.DMA,
            pltpu.SemaphoreType.DMA,
            pltpu.SemaphoreType.DMA,
            pltpu.SemaphoreType.DMA,
        ],
        compiler_params=pltpu.CompilerParams(needs_layout_passes=False),
    )(_spmm_body)


def _spmm_body(h_hbm, col_hbm, row_hbm, out_hbm, cstage, rstage, buf, aggs,
               semc, semr, semg, sems):
    c = lax.axis_index("c")
    s = lax.axis_index("s")
    wid = c * NS + s

    zeros16 = jnp.zeros((L,), jnp.float32)

    def zbody(r, carry):
        for nb in range(NBUF):
            for l in range(D // L):
                buf[nb, r, pl.ds(l * L, L)] = zeros16
        return carry

    lax.fori_loop(0, K, zbody, 0, unroll=2)
    for j in range(RPT // K):
        pltpu.sync_copy(buf.at[0], aggs.at[pl.ds(s * RPT + j * K, K)])
    plsc.subcore_barrier()

    pltpu.async_copy(
        col_hbm.at[wid, pl.ds(0, CHB)], cstage.at[0], semc).wait()
    pltpu.async_copy(
        row_hbm.at[wid, pl.ds(0, CHB)], rstage.at[0], semr).wait()
    for i in range(LA):
        pltpu.async_copy(
            buf.at[LA + i], aggs.at[rstage.at[0, 0]], sems, add=True)
        pltpu.async_copy(h_hbm.at[cstage.at[0, i]], buf.at[i], semg)

    def chunk_body(q, carry):
        a = lax.rem(q, 2)
        na = 1 - a
        nxt = (q + 1) * CHB
        pltpu.async_copy(
            col_hbm.at[wid, pl.ds(nxt, CHB)], cstage.at[na], semc)
        pltpu.async_copy(
            row_hbm.at[wid, pl.ds(nxt, CHB)], rstage.at[na], semr)
        for b in range(CHB):
            sl = b % NBUF
            pltpu.make_async_copy(
                h_hbm.at[cstage.at[a, b]], buf.at[sl], semg).wait()
            pltpu.async_copy(
                buf.at[sl], aggs.at[rstage.at[a, b]], sems, add=True)
            pltpu.make_async_copy(
                buf.at[sl], aggs.at[rstage.at[a, b]], sems).wait()
            nb = b + LA
            if nb < CHB:
                pltpu.async_copy(
                    h_hbm.at[cstage.at[a, nb]], buf.at[nb % NBUF], semg)
            else:
                if nb == CHB:
                    pltpu.make_async_copy(
                        col_hbm.at[wid, pl.ds(nxt, CHB)], cstage.at[na],
                        semc).wait()
                    pltpu.make_async_copy(
                        row_hbm.at[wid, pl.ds(nxt, CHB)], rstage.at[na],
                        semr).wait()
                pltpu.async_copy(
                    h_hbm.at[cstage.at[na, nb - CHB]], buf.at[nb % NBUF],
                    semg)
        return carry

    lax.fori_loop(0, NCH, chunk_body, 0)
    for i in range(LA):
        pltpu.make_async_copy(
            h_hbm.at[cstage.at[0, 0]], buf.at[i], semg).wait()
        pltpu.make_async_copy(
            buf.at[i], aggs.at[rstage.at[0, 0]], sems).wait()

    plsc.subcore_barrier()
    pltpu.sync_copy(aggs.at[pl.ds(s * RPT, RPT)],
                    out_hbm.at[c, pl.ds(s * RPT, RPT)])


_BN = 128


def _dinv_rows(degp):
    d = jnp.sum(degp, axis=0)
    dinv = jnp.where(d > 0, lax.rsqrt(d), 0.0)
    rows = jnp.broadcast_to(dinv[None, :], (_BN, _BN))
    eye = (lax.broadcasted_iota(jnp.int32, (_BN, _BN), 0)
           == lax.broadcasted_iota(jnp.int32, (_BN, _BN), 1))
    dm = jnp.where(eye, rows, 0.0)
    ones = jnp.ones((_BN, D), jnp.float32)
    return jnp.dot(dm, ones, preferred_element_type=jnp.float32)


def _l1_body(x_ref, degp_ref, w1t_ref, wr1t_ref, b1_ref, br1_ref,
             h_ref, r_ref, dv_ref):
    dinv_b = _dinv_rows(degp_ref[...].reshape(NW, 128))
    x = x_ref[...]
    t = jnp.dot(x, w1t_ref[...], preferred_element_type=jnp.float32) + b1_ref[...]
    h_ref[...] = dinv_b * t
    r_ref[...] = jnp.dot(x, wr1t_ref[...],
                         preferred_element_type=jnp.float32) + br1_ref[...]
    dv_ref[...] = dinv_b


def _mid_body(aggp_ref, dv_ref, r1_ref, w2t_ref, wr2t_ref, b2_ref, br2_ref,
              x1_ref, h2_ref, r2_ref):
    dinv_b = dv_ref[...]
    a = aggp_ref[0] + aggp_ref[1]
    x1 = jnp.maximum(dinv_b * a + r1_ref[...], 0.0)
    x1_ref[...] = x1
    t = jnp.dot(x1, w2t_ref[...], preferred_element_type=jnp.float32) + b2_ref[...]
    h2_ref[...] = dinv_b * t
    r2_ref[...] = jnp.dot(x1, wr2t_ref[...],
                          preferred_element_type=jnp.float32) + br2_ref[...]


def _fin_body(aggp_ref, dv_ref, r2_ref, x1_ref, wlt_ref, bl_ref, out_ref):
    a = aggp_ref[0] + aggp_ref[1]
    x2 = jnp.maximum(dv_ref[...] * a + r2_ref[...], 0.0)
    xc = jnp.concatenate([x1_ref[...], x2], axis=1)
    t = jnp.dot(xc, wlt_ref[...], preferred_element_type=jnp.float32) + bl_ref[...]
    m = jnp.max(t, axis=1, keepdims=True)
    lse = jnp.log(jnp.sum(jnp.exp(t - m), axis=1, keepdims=True))
    out_ref[...] = t - m - lse


def _row_spec(width):
    return pl.BlockSpec((_BN, width), lambda i: (i, 0))


def _full_spec(shape):
    nd = len(shape)
    return pl.BlockSpec(shape, lambda i, _n=nd: (0,) * _n)


_l1_call = pl.pallas_call(
    _l1_body,
    grid=(NPAD // _BN,),
    in_specs=[
        _row_spec(D),
        pl.BlockSpec((1, NW, 128), lambda i: (i, 0, 0)),
        _full_spec((D, D)), _full_spec((D, D)),
        _full_spec((1, D)), _full_spec((1, D)),
    ],
    out_specs=[_row_spec(D), _row_spec(D), _row_spec(D)],
    out_shape=[jax.ShapeDtypeStruct((NPAD, D), jnp.float32)] * 3,
)

_mid_call = pl.pallas_call(
    _mid_body,
    grid=(NPAD // _BN,),
    in_specs=[
        pl.BlockSpec((NC, _BN, D), lambda i: (0, i, 0)),
        _row_spec(D), _row_spec(D),
        _full_spec((D, D)), _full_spec((D, D)),
        _full_spec((1, D)), _full_spec((1, D)),
    ],
    out_specs=[_row_spec(D), _row_spec(D), _row_spec(D)],
    out_shape=[jax.ShapeDtypeStruct((NPAD, D), jnp.float32)] * 3,
)

_fin_call = pl.pallas_call(
    _fin_body,
    grid=(-(-N // _BN),),
    in_specs=[
        pl.BlockSpec((NC, _BN, D), lambda i: (0, i, 0)),
        _row_spec(D), _row_spec(D), _row_spec(D),
        _full_spec((2 * D, C)), _full_spec((1, C)),
    ],
    out_specs=pl.BlockSpec((_BN, C), lambda i: (i, 0)),
    out_shape=jax.ShapeDtypeStruct((N, C), jnp.float32),
)


def kernel(x0, edge_index, W1, b1, Wr1, br1, W2, b2, Wr2, br2, Wl, bl):
    row = edge_index[0]
    col = edge_index[1]
    npads = EPAD - E
    pad_ids = N + lax.iota(jnp.int32, npads) % NDUM
    rowp = jnp.concatenate([row, pad_ids])
    colp = jnp.concatenate([col, pad_ids])
    dumd = (N + lax.iota(jnp.int32, NW * CHD) % NDUM).reshape(NW, CHD)
    row_flat = jnp.concatenate(
        [rowp.reshape(NW, EPT), dumd], axis=1).reshape(NW, EPT // 128 + CHDR,
                                                       128)
    dumb = (N + lax.iota(jnp.int32, NW * CHB * K) % NDUM).reshape(NW, CHB, K)
    row3 = jnp.concatenate([rowp.reshape(NW, BLK, K), dumb], axis=1)
    col3 = jnp.concatenate([colp.reshape(NW, BLK, K), dumb], axis=1)

    x0p = jnp.pad(x0, ((0, NPAD - N), (0, 0)))
    b1r = b1.reshape(1, D)
    br1r = br1.reshape(1, D)
    b2r = b2.reshape(1, D)
    br2r = br2.reshape(1, D)
    blr = bl.reshape(1, C)

    degp = jnp.transpose(_deg_kernel(row_flat), (1, 0, 2))
    h1p, r1p, dinv_b = _l1_call(x0p, degp, W1.T, Wr1.T, b1r, br1r)
    agg1 = _spmm_kernel(h1p, col3, row3)
    x1, h2p, r2p = _mid_call(agg1, dinv_b, r1p, W2.T, Wr2.T, b2r, br2r)
    agg2 = _spmm_kernel(h2p, col3, row3)
    return _fin_call(agg2, dinv_b, r2p, x1, Wl.T, blr)


def _deg_kernel(rows_hbm):
    return _build_deg_kernel()(rows_hbm)


def _spmm_kernel(h, col3, row3):
    return _build_spmm_kernel()(h, col3, row3)


# speedup vs baseline: 25.1799x; 1.2950x over previous
import functools

import jax
import jax.numpy as jnp
from jax import lax
from jax.experimental import pallas as pl
from jax.experimental.pallas import tpu as pltpu
from jax.experimental.pallas import tpu_sc as plsc

N = 10000
D = 128
C = 64
NPAD = 10240
NDUM = NPAD - N
NC = 2
NS = 16
NW = NC * NS
L = 16
E = 320000
K = 64
BLK = 160
EPT = BLK * K
EPAD = NW * EPT
RPT = NPAD // NS
CHB = 8
NCH = BLK // CHB
NBUF = 4
LA = 2
CHD = 2048
NCHD = EPT // CHD
CHDR = CHD // 128
NR = NPAD // 128

@functools.cache
def _build_deg_kernel():
    mesh = plsc.VectorSubcoreMesh(
        core_axis_name="c", subcore_axis_name="s",
        num_cores=NC, num_subcores=NS)
    return functools.partial(
        pl.kernel,
        out_type=jax.ShapeDtypeStruct((NW, NR, 128), jnp.float32),
        mesh=mesh,
        scratch_types=[
            pltpu.VMEM((2, CHDR, 128), jnp.int32),
            pltpu.VMEM((NR, 128), jnp.float32),
            pltpu.SemaphoreType.DMA,
        ],
        compiler_params=pltpu.CompilerParams(needs_layout_passes=False),
    )(_deg_body)


def _deg_body(rows_hbm, out_hbm, stage, degv, semd):
    c = lax.axis_index("c")
    s = lax.axis_index("s")
    wid = c * NS + s
    zeros16 = jnp.zeros((L,), jnp.float32)
    ones16 = jnp.ones((L,), jnp.float32)

    def zbody(r, carry):
        for l in range(128 // L):
            degv[r, pl.ds(l * L, L)] = zeros16
        return carry

    lax.fori_loop(0, NR, zbody, 0, unroll=2)
    pltpu.async_copy(
        rows_hbm.at[wid, pl.ds(0, CHDR)], stage.at[0], semd).wait()

    def chunk(q, carry):
        a = lax.rem(q, 2)
        na = 1 - a
        nxt = (q + 1) * CHDR
        pltpu.async_copy(rows_hbm.at[wid, pl.ds(nxt, CHDR)], stage.at[na], semd)

        def body(i, carry2):
            r = lax.shift_right_logical(i, 3)
            l = lax.rem(i, 8)
            idx = stage[a, r, pl.ds(l * L, L)]
            hi = lax.shift_right_logical(idx, 7)
            lo = lax.rem(idx, 128)
            plsc.addupdate_scatter(degv, [hi, lo], ones16)
            return carry2

        lax.fori_loop(0, CHD // L, body, 0, unroll=4)
        pltpu.make_async_copy(
            rows_hbm.at[wid, pl.ds(nxt, CHDR)], stage.at[na], semd).wait()
        return carry

    lax.fori_loop(0, NCHD, chunk, 0)
    pltpu.sync_copy(degv, out_hbm.at[wid])


@functools.cache
def _build_spmm_kernel():
    mesh = plsc.VectorSubcoreMesh(
        core_axis_name="c", subcore_axis_name="s",
        num_cores=NC, num_subcores=NS)
    return functools.partial(
        pl.kernel,
        out_type=jax.ShapeDtypeStruct((NC, NPAD, D), jnp.float32),
        mesh=mesh,
        scratch_types=[
            pltpu.VMEM((2, CHB, K), jnp.int32),
            pltpu.VMEM((2, CHB, K), jnp.int32),
            pltpu.VMEM((NBUF, K, D), jnp.float32),
            pltpu.VMEM_SHARED((NPAD, D), jnp.float32),
            pltpu.SemaphoreType.DMA,
            pltpu.SemaphoreType.DMA,
            pltpu.SemaphoreType.DMA,
            pltpu.SemaphoreType.DMA,
        ],
        compiler_params=pltpu.CompilerParams(needs_layout_passes=False),
    )(_spmm_body)


def _spmm_body(h_hbm, col_hbm, row_hbm, out_hbm, cstage, rstage, buf, aggs,
               semc, semr, semg, sems):
    c = lax.axis_index("c")
    s = lax.axis_index("s")
    wid = c * NS + s

    zeros16 = jnp.zeros((L,), jnp.float32)

    def zbody(r, carry):
        for nb in range(NBUF):
            for l in range(D // L):
                buf[nb, r, pl.ds(l * L, L)] = zeros16
        return carry

    lax.fori_loop(0, K, zbody, 0, unroll=2)
    for j in range(RPT // K):
        pltpu.sync_copy(buf.at[0], aggs.at[pl.ds(s * RPT + j * K, K)])
    plsc.subcore_barrier()

    pltpu.async_copy(
        col_hbm.at[wid, pl.ds(0, CHB)], cstage.at[0], semc).wait()
    pltpu.async_copy(
        row_hbm.at[wid, pl.ds(0, CHB)], rstage.at[0], semr).wait()
    for i in range(LA):
        pltpu.async_copy(
            buf.at[LA + i], aggs.at[rstage.at[0, 0]], sems, add=True)
        pltpu.async_copy(h_hbm.at[cstage.at[0, i]], buf.at[i], semg)

    def chunk_body(q, carry):
        a = lax.rem(q, 2)
        na = 1 - a
        nxt = (q + 1) * CHB
        pltpu.async_copy(
            col_hbm.at[wid, pl.ds(nxt, CHB)], cstage.at[na], semc)
        pltpu.async_copy(
            row_hbm.at[wid, pl.ds(nxt, CHB)], rstage.at[na], semr)
        for b in range(CHB):
            sl = b % NBUF
            pltpu.make_async_copy(
                h_hbm.at[cstage.at[a, b]], buf.at[sl], semg).wait()
            pltpu.async_copy(
                buf.at[sl], aggs.at[rstage.at[a, b]], sems, add=True)
            pltpu.make_async_copy(
                buf.at[sl], aggs.at[rstage.at[a, b]], sems).wait()
            nb = b + LA
            if nb < CHB:
                pltpu.async_copy(
                    h_hbm.at[cstage.at[a, nb]], buf.at[nb % NBUF], semg)
            else:
                if nb == CHB:
                    pltpu.make_async_copy(
                        col_hbm.at[wid, pl.ds(nxt, CHB)], cstage.at[na],
                        semc).wait()
                    pltpu.make_async_copy(
                        row_hbm.at[wid, pl.ds(nxt, CHB)], rstage.at[na],
                        semr).wait()
                pltpu.async_copy(
                    h_hbm.at[cstage.at[na, nb - CHB]], buf.at[nb % NBUF],
                    semg)
        return carry

    lax.fori_loop(0, NCH, chunk_body, 0)
    for i in range(LA):
        pltpu.make_async_copy(
            h_hbm.at[cstage.at[0, 0]], buf.at[i], semg).wait()
        pltpu.make_async_copy(
            buf.at[i], aggs.at[rstage.at[0, 0]], sems).wait()

    plsc.subcore_barrier()
    pltpu.sync_copy(aggs.at[pl.ds(s * RPT, RPT)],
                    out_hbm.at[c, pl.ds(s * RPT, RPT)])


_BN = 512
_BG = _BN // 128


def _dinv_rows(degp):
    d = jnp.sum(degp, axis=1)
    dinv = jnp.where(d > 0, lax.rsqrt(d), 0.0)
    eye = (lax.broadcasted_iota(jnp.int32, (128, 128), 0)
           == lax.broadcasted_iota(jnp.int32, (128, 128), 1))
    ones = jnp.ones((128, D), jnp.float32)
    groups = [
        jnp.dot(jnp.where(eye, dinv[g][None, :], 0.0), ones,
                preferred_element_type=jnp.float32)
        for g in range(_BG)
    ]
    return jnp.concatenate(groups, axis=0)


def _l1_body(x_ref, degp_ref, w1t_ref, wr1t_ref, b1_ref, br1_ref,
             h_ref, r_ref):
    dinv_b = _dinv_rows(degp_ref[...])
    x = x_ref[...]
    t = jnp.dot(x, w1t_ref[...], preferred_element_type=jnp.float32) + b1_ref[...]
    h_ref[...] = dinv_b * t
    r_ref[...] = jnp.dot(x, wr1t_ref[...],
                         preferred_element_type=jnp.float32) + br1_ref[...]


def _mid_body(aggp_ref, degp_ref, r1_ref, w2t_ref, wr2t_ref, b2_ref, br2_ref,
              x1_ref, h2_ref, r2_ref):
    dinv_b = _dinv_rows(degp_ref[...])
    a = aggp_ref[0] + aggp_ref[1]
    x1 = jnp.maximum(dinv_b * a + r1_ref[...], 0.0)
    x1_ref[...] = x1
    t = jnp.dot(x1, w2t_ref[...], preferred_element_type=jnp.float32) + b2_ref[...]
    h2_ref[...] = dinv_b * t
    r2_ref[...] = jnp.dot(x1, wr2t_ref[...],
                          preferred_element_type=jnp.float32) + br2_ref[...]


def _fin_body(aggp_ref, degp_ref, r2_ref, x1_ref, wlt_ref, bl_ref, out_ref):
    dinv_b = _dinv_rows(degp_ref[...])
    a = aggp_ref[0] + aggp_ref[1]
    x2 = jnp.maximum(dinv_b * a + r2_ref[...], 0.0)
    xc = jnp.concatenate([x1_ref[...], x2], axis=1)
    t = jnp.dot(xc, wlt_ref[...], preferred_element_type=jnp.float32) + bl_ref[...]
    m = jnp.max(t, axis=1, keepdims=True)
    lse = jnp.log(jnp.sum(jnp.exp(t - m), axis=1, keepdims=True))
    out_ref[...] = t - m - lse


def _row_spec(width):
    return pl.BlockSpec((_BN, width), lambda i: (i, 0))


def _degp_spec():
    return pl.BlockSpec((_BG, NW, 128), lambda i: (i, 0, 0))


def _full_spec(shape):
    nd = len(shape)
    return pl.BlockSpec(shape, lambda i, _n=nd: (0,) * _n)


_l1_call = pl.pallas_call(
    _l1_body,
    grid=(NPAD // _BN,),
    in_specs=[
        _row_spec(D),
        _degp_spec(),
        _full_spec((D, D)), _full_spec((D, D)),
        _full_spec((1, D)), _full_spec((1, D)),
    ],
    out_specs=[_row_spec(D), _row_spec(D)],
    out_shape=[jax.ShapeDtypeStruct((NPAD, D), jnp.float32)] * 2,
)

_mid_call = pl.pallas_call(
    _mid_body,
    grid=(NPAD // _BN,),
    in_specs=[
        pl.BlockSpec((NC, _BN, D), lambda i: (0, i, 0)),
        _degp_spec(), _row_spec(D),
        _full_spec((D, D)), _full_spec((D, D)),
        _full_spec((1, D)), _full_spec((1, D)),
    ],
    out_specs=[_row_spec(D), _row_spec(D), _row_spec(D)],
    out_shape=[jax.ShapeDtypeStruct((NPAD, D), jnp.float32)] * 3,
)

_fin_call = pl.pallas_call(
    _fin_body,
    grid=(NPAD // _BN,),
    in_specs=[
        pl.BlockSpec((NC, _BN, D), lambda i: (0, i, 0)),
        _degp_spec(), _row_spec(D), _row_spec(D),
        _full_spec((2 * D, C)), _full_spec((1, C)),
    ],
    out_specs=pl.BlockSpec((_BN, C), lambda i: (i, 0)),
    out_shape=jax.ShapeDtypeStruct((NPAD, C), jnp.float32),
)


def kernel(x0, edge_index, W1, b1, Wr1, br1, W2, b2, Wr2, br2, Wl, bl):
    row = edge_index[0]
    col = edge_index[1]
    npads = EPAD - E
    pad_ids = N + lax.iota(jnp.int32, npads) % NDUM
    rowp = jnp.concatenate([row, pad_ids])
    colp = jnp.concatenate([col, pad_ids])
    dumd = (N + lax.iota(jnp.int32, NW * CHD) % NDUM).reshape(NW, CHD)
    row_flat = jnp.concatenate(
        [rowp.reshape(NW, EPT), dumd], axis=1).reshape(NW, EPT // 128 + CHDR,
                                                       128)
    dumb = (N + lax.iota(jnp.int32, NW * CHB * K) % NDUM).reshape(NW, CHB, K)
    row3 = jnp.concatenate([rowp.reshape(NW, BLK, K), dumb], axis=1)
    col3 = jnp.concatenate([colp.reshape(NW, BLK, K), dumb], axis=1)

    x0p = jnp.pad(x0, ((0, NPAD - N), (0, 0)))
    b1r = b1.reshape(1, D)
    br1r = br1.reshape(1, D)
    b2r = b2.reshape(1, D)
    br2r = br2.reshape(1, D)
    blr = bl.reshape(1, C)

    degp = jnp.transpose(_deg_kernel(row_flat), (1, 0, 2))
    h1p, r1p = _l1_call(x0p, degp, W1.T, Wr1.T, b1r, br1r)
    agg1 = _spmm_kernel(h1p, col3, row3)
    x1, h2p, r2p = _mid_call(agg1, degp, r1p, W2.T, Wr2.T, b2r, br2r)
    agg2 = _spmm_kernel(h2p, col3, row3)
    return _fin_call(agg2, degp, r2p, x1, Wl.T, blr)[:N]


def _deg_kernel(rows_hbm):
    return _build_deg_kernel()(rows_hbm)


def _spmm_kernel(h, col3, row3):
    return _build_spmm_kernel()(h, col3, row3)


# speedup vs baseline: 26.7127x; 1.0609x over previous
import functools

import jax
import jax.numpy as jnp
from jax import lax
from jax.experimental import pallas as pl
from jax.experimental.pallas import tpu as pltpu
from jax.experimental.pallas import tpu_sc as plsc

N = 10000
D = 128
C = 64
NPAD = 10240
NDUM = NPAD - N
NC = 2
NS = 16
NW = NC * NS
L = 16
E = 320000
K = 64
BLK = 160
EPT = BLK * K
EPAD = NW * EPT
RPT = NPAD // NS
CHB = 8
NCH = BLK // CHB
NBUF = 4
LA = 2
CHD = 2048
NCHD = EPT // CHD
CHDR = CHD // 128
NR = NPAD // 128

@functools.cache
def _build_deg_kernel():
    mesh = plsc.VectorSubcoreMesh(
        core_axis_name="c", subcore_axis_name="s",
        num_cores=NC, num_subcores=NS)
    return functools.partial(
        pl.kernel,
        out_type=jax.ShapeDtypeStruct((NW, NR, 128), jnp.float32),
        mesh=mesh,
        scratch_types=[
            pltpu.VMEM((2, CHDR, 128), jnp.int32),
            pltpu.VMEM((NR, 128), jnp.float32),
            pltpu.SemaphoreType.DMA,
        ],
        compiler_params=pltpu.CompilerParams(needs_layout_passes=False),
    )(_deg_body)


def _deg_body(rows_hbm, out_hbm, stage, degv, semd):
    c = lax.axis_index("c")
    s = lax.axis_index("s")
    wid = c * NS + s
    zeros16 = jnp.zeros((L,), jnp.float32)
    ones16 = jnp.ones((L,), jnp.float32)

    def zbody(r, carry):
        for l in range(128 // L):
            degv[r, pl.ds(l * L, L)] = zeros16
        return carry

    lax.fori_loop(0, NR, zbody, 0, unroll=2)
    pltpu.async_copy(
        rows_hbm.at[wid, pl.ds(0, CHDR)], stage.at[0], semd).wait()

    def chunk(q, carry):
        a = lax.rem(q, 2)
        na = 1 - a
        nxt = (q + 1) * CHDR
        pltpu.async_copy(rows_hbm.at[wid, pl.ds(nxt, CHDR)], stage.at[na], semd)

        def body(i, carry2):
            r = lax.shift_right_logical(i, 3)
            l = lax.rem(i, 8)
            idx = stage[a, r, pl.ds(l * L, L)]
            hi = lax.shift_right_logical(idx, 7)
            lo = lax.rem(idx, 128)
            plsc.addupdate_scatter(degv, [hi, lo], ones16)
            return carry2

        lax.fori_loop(0, CHD // L, body, 0, unroll=4)
        pltpu.make_async_copy(
            rows_hbm.at[wid, pl.ds(nxt, CHDR)], stage.at[na], semd).wait()
        return carry

    lax.fori_loop(0, NCHD, chunk, 0)
    pltpu.sync_copy(degv, out_hbm.at[wid])


@functools.cache
def _build_spmm_kernel():
    mesh = plsc.VectorSubcoreMesh(
        core_axis_name="c", subcore_axis_name="s",
        num_cores=NC, num_subcores=NS)
    return functools.partial(
        pl.kernel,
        out_type=jax.ShapeDtypeStruct((NC, NPAD, D), jnp.float32),
        mesh=mesh,
        scratch_types=[
            pltpu.VMEM((2, CHB, K), jnp.int32),
            pltpu.VMEM((2, CHB, K), jnp.int32),
            pltpu.VMEM((NBUF, K, D), jnp.float32),
            pltpu.VMEM_SHARED((NPAD, D), jnp.float32),
            pltpu.SemaphoreType.DMA,
            pltpu.SemaphoreType.DMA,
            pltpu.SemaphoreType.DMA,
            pltpu.SemaphoreType.DMA,
        ],
        compiler_params=pltpu.CompilerParams(needs_layout_passes=False),
    )(_spmm_body)


def _spmm_body(h_hbm, col_hbm, row_hbm, out_hbm, cstage, rstage, buf, aggs,
               semc, semr, semg, sems):
    c = lax.axis_index("c")
    s = lax.axis_index("s")
    wid = c * NS + s

    zeros16 = jnp.zeros((L,), jnp.float32)

    def zbody(r, carry):
        for nb in range(NBUF):
            for l in range(D // L):
                buf[nb, r, pl.ds(l * L, L)] = zeros16
        return carry

    lax.fori_loop(0, K, zbody, 0, unroll=2)
    for j in range(RPT // K):
        pltpu.sync_copy(buf.at[0], aggs.at[pl.ds(s * RPT + j * K, K)])
    plsc.subcore_barrier()

    pltpu.async_copy(
        col_hbm.at[wid, pl.ds(0, CHB)], cstage.at[0], semc).wait()
    pltpu.async_copy(
        row_hbm.at[wid, pl.ds(0, CHB)], rstage.at[0], semr).wait()
    for i in range(LA):
        pltpu.async_copy(
            buf.at[LA + i], aggs.at[rstage.at[0, 0]], sems, add=True)
        pltpu.async_copy(h_hbm.at[cstage.at[0, i]], buf.at[i], semg)

    def chunk_body(q, carry):
        a = lax.rem(q, 2)
        na = 1 - a
        nxt = (q + 1) * CHB
        pltpu.async_copy(
            col_hbm.at[wid, pl.ds(nxt, CHB)], cstage.at[na], semc)
        pltpu.async_copy(
            row_hbm.at[wid, pl.ds(nxt, CHB)], rstage.at[na], semr)
        for b in range(CHB):
            sl = b % NBUF
            pltpu.make_async_copy(
                h_hbm.at[cstage.at[a, b]], buf.at[sl], semg).wait()
            pltpu.async_copy(
                buf.at[sl], aggs.at[rstage.at[a, b]], sems, add=True)
            pltpu.make_async_copy(
                buf.at[sl], aggs.at[rstage.at[a, b]], sems).wait()
            nb = b + LA
            if nb < CHB:
                pltpu.async_copy(
                    h_hbm.at[cstage.at[a, nb]], buf.at[nb % NBUF], semg)
            else:
                if nb == CHB:
                    pltpu.make_async_copy(
                        col_hbm.at[wid, pl.ds(nxt, CHB)], cstage.at[na],
                        semc).wait()
                    pltpu.make_async_copy(
                        row_hbm.at[wid, pl.ds(nxt, CHB)], rstage.at[na],
                        semr).wait()
                pltpu.async_copy(
                    h_hbm.at[cstage.at[na, nb - CHB]], buf.at[nb % NBUF],
                    semg)
        return carry

    lax.fori_loop(0, NCH, chunk_body, 0)
    for i in range(LA):
        pltpu.make_async_copy(
            h_hbm.at[cstage.at[0, 0]], buf.at[i], semg).wait()
        pltpu.make_async_copy(
            buf.at[i], aggs.at[rstage.at[0, 0]], sems).wait()

    plsc.subcore_barrier()
    pltpu.sync_copy(aggs.at[pl.ds(s * RPT, RPT)],
                    out_hbm.at[c, pl.ds(s * RPT, RPT)])


_BN = 1024
_BG = _BN // 128


def _dinv_rows(degp):
    d = jnp.sum(degp, axis=1)
    dinv = jnp.where(d > 0, lax.rsqrt(d), 0.0)
    eye = (lax.broadcasted_iota(jnp.int32, (128, 128), 0)
           == lax.broadcasted_iota(jnp.int32, (128, 128), 1))
    ones = jnp.ones((128, D), jnp.float32)
    groups = [
        jnp.dot(jnp.where(eye, dinv[g][None, :], 0.0), ones,
                preferred_element_type=jnp.float32)
        for g in range(_BG)
    ]
    return jnp.concatenate(groups, axis=0)


def _l1_body(x_ref, degp_ref, w1t_ref, wr1t_ref, b1_ref, br1_ref,
             h_ref, r_ref):
    dinv_b = _dinv_rows(degp_ref[...])
    x = x_ref[...]
    t = jnp.dot(x, w1t_ref[...], preferred_element_type=jnp.float32) + b1_ref[...]
    h_ref[...] = dinv_b * t
    r_ref[...] = jnp.dot(x, wr1t_ref[...],
                         preferred_element_type=jnp.float32) + br1_ref[...]


def _mid_body(aggp_ref, degp_ref, r1_ref, w2t_ref, wr2t_ref, b2_ref, br2_ref,
              x1_ref, h2_ref, r2_ref):
    dinv_b = _dinv_rows(degp_ref[...])
    a = aggp_ref[0] + aggp_ref[1]
    x1 = jnp.maximum(dinv_b * a + r1_ref[...], 0.0)
    x1_ref[...] = x1
    t = jnp.dot(x1, w2t_ref[...], preferred_element_type=jnp.float32) + b2_ref[...]
    h2_ref[...] = dinv_b * t
    r2_ref[...] = jnp.dot(x1, wr2t_ref[...],
                          preferred_element_type=jnp.float32) + br2_ref[...]


def _fin_body(aggp_ref, degp_ref, r2_ref, x1_ref, wlt_ref, bl_ref, out_ref):
    dinv_b = _dinv_rows(degp_ref[...])
    a = aggp_ref[0] + aggp_ref[1]
    x2 = jnp.maximum(dinv_b * a + r2_ref[...], 0.0)
    xc = jnp.concatenate([x1_ref[...], x2], axis=1)
    t = jnp.dot(xc, wlt_ref[...], preferred_element_type=jnp.float32) + bl_ref[...]
    m = jnp.max(t, axis=1, keepdims=True)
    lse = jnp.log(jnp.sum(jnp.exp(t - m), axis=1, keepdims=True))
    out_ref[...] = t - m - lse


def _row_spec(width):
    return pl.BlockSpec((_BN, width), lambda i: (i, 0))


def _degp_spec():
    return pl.BlockSpec((_BG, NW, 128), lambda i: (i, 0, 0))


def _full_spec(shape):
    nd = len(shape)
    return pl.BlockSpec(shape, lambda i, _n=nd: (0,) * _n)


_l1_call = pl.pallas_call(
    _l1_body,
    grid=(NPAD // _BN,),
    in_specs=[
        _row_spec(D),
        _degp_spec(),
        _full_spec((D, D)), _full_spec((D, D)),
        _full_spec((1, D)), _full_spec((1, D)),
    ],
    out_specs=[_row_spec(D), _row_spec(D)],
    out_shape=[jax.ShapeDtypeStruct((NPAD, D), jnp.float32)] * 2,
)

_mid_call = pl.pallas_call(
    _mid_body,
    grid=(NPAD // _BN,),
    in_specs=[
        pl.BlockSpec((NC, _BN, D), lambda i: (0, i, 0)),
        _degp_spec(), _row_spec(D),
        _full_spec((D, D)), _full_spec((D, D)),
        _full_spec((1, D)), _full_spec((1, D)),
    ],
    out_specs=[_row_spec(D), _row_spec(D), _row_spec(D)],
    out_shape=[jax.ShapeDtypeStruct((NPAD, D), jnp.float32)] * 3,
)

_fin_call = pl.pallas_call(
    _fin_body,
    grid=(NPAD // _BN,),
    in_specs=[
        pl.BlockSpec((NC, _BN, D), lambda i: (0, i, 0)),
        _degp_spec(), _row_spec(D), _row_spec(D),
        _full_spec((2 * D, C)), _full_spec((1, C)),
    ],
    out_specs=pl.BlockSpec((_BN, C), lambda i: (i, 0)),
    out_shape=jax.ShapeDtypeStruct((NPAD, C), jnp.float32),
)


def kernel(x0, edge_index, W1, b1, Wr1, br1, W2, b2, Wr2, br2, Wl, bl):
    row = edge_index[0]
    col = edge_index[1]
    npads = EPAD - E
    pad_ids = N + lax.iota(jnp.int32, npads) % NDUM
    rowp = jnp.concatenate([row, pad_ids])
    colp = jnp.concatenate([col, pad_ids])
    dumd = (N + lax.iota(jnp.int32, NW * CHD) % NDUM).reshape(NW, CHD)
    row_flat = jnp.concatenate(
        [rowp.reshape(NW, EPT), dumd], axis=1).reshape(NW, EPT // 128 + CHDR,
                                                       128)
    dumb = (N + lax.iota(jnp.int32, NW * CHB * K) % NDUM).reshape(NW, CHB, K)
    row3 = jnp.concatenate([rowp.reshape(NW, BLK, K), dumb], axis=1)
    col3 = jnp.concatenate([colp.reshape(NW, BLK, K), dumb], axis=1)

    x0p = jnp.pad(x0, ((0, NPAD - N), (0, 0)))
    b1r = b1.reshape(1, D)
    br1r = br1.reshape(1, D)
    b2r = b2.reshape(1, D)
    br2r = br2.reshape(1, D)
    blr = bl.reshape(1, C)

    degp = jnp.transpose(_deg_kernel(row_flat), (1, 0, 2))
    h1p, r1p = _l1_call(x0p, degp, W1.T, Wr1.T, b1r, br1r)
    agg1 = _spmm_kernel(h1p, col3, row3)
    x1, h2p, r2p = _mid_call(agg1, degp, r1p, W2.T, Wr2.T, b2r, br2r)
    agg2 = _spmm_kernel(h2p, col3, row3)
    return _fin_call(agg2, degp, r2p, x1, Wl.T, blr)[:N]


def _deg_kernel(rows_hbm):
    return _build_deg_kernel()(rows_hbm)


def _spmm_kernel(h, col3, row3):
    return _build_spmm_kernel()(h, col3, row3)


# speedup vs baseline: 27.3790x; 1.0249x over previous
import functools

import jax
import jax.numpy as jnp
from jax import lax
from jax.experimental import pallas as pl
from jax.experimental.pallas import tpu as pltpu
from jax.experimental.pallas import tpu_sc as plsc

N = 10000
D = 128
C = 64
NPAD = 10240
NDUM = NPAD - N
NC = 2
NS = 16
NW = NC * NS
L = 16
E = 320000
K = 64
BLK = 160
EPT = BLK * K
EPAD = NW * EPT
RPT = NPAD // NS
CHB = 8
NCH = BLK // CHB
NBUF = 4
LA = 2
CHD = 2048
NCHD = EPT // CHD
CHDR = CHD // 128
NR = NPAD // 128

@functools.cache
def _build_deg_kernel():
    mesh = plsc.VectorSubcoreMesh(
        core_axis_name="c", subcore_axis_name="s",
        num_cores=NC, num_subcores=NS)
    return functools.partial(
        pl.kernel,
        out_type=jax.ShapeDtypeStruct((NW, NR, 128), jnp.float32),
        mesh=mesh,
        scratch_types=[
            pltpu.VMEM((2, CHDR, 128), jnp.int32),
            pltpu.VMEM((NR, 128), jnp.float32),
            pltpu.SemaphoreType.DMA,
        ],
        compiler_params=pltpu.CompilerParams(needs_layout_passes=False),
    )(_deg_body)


def _deg_body(rows_hbm, out_hbm, stage, degv, semd):
    c = lax.axis_index("c")
    s = lax.axis_index("s")
    wid = c * NS + s
    zeros16 = jnp.zeros((L,), jnp.float32)
    ones16 = jnp.ones((L,), jnp.float32)

    def zbody(r, carry):
        for l in range(128 // L):
            degv[r, pl.ds(l * L, L)] = zeros16
        return carry

    lax.fori_loop(0, NR, zbody, 0, unroll=2)
    pltpu.async_copy(
        rows_hbm.at[wid, pl.ds(0, CHDR)], stage.at[0], semd).wait()

    def chunk(q, carry):
        a = lax.rem(q, 2)
        na = 1 - a
        nxt = (q + 1) * CHDR
        pltpu.async_copy(rows_hbm.at[wid, pl.ds(nxt, CHDR)], stage.at[na], semd)

        def body(i, carry2):
            r = lax.shift_right_logical(i, 3)
            l = lax.rem(i, 8)
            idx = stage[a, r, pl.ds(l * L, L)]
            hi = lax.shift_right_logical(idx, 7)
            lo = lax.rem(idx, 128)
            plsc.addupdate_scatter(degv, [hi, lo], ones16)
            return carry2

        lax.fori_loop(0, CHD // L, body, 0, unroll=4)
        pltpu.make_async_copy(
            rows_hbm.at[wid, pl.ds(nxt, CHDR)], stage.at[na], semd).wait()
        return carry

    lax.fori_loop(0, NCHD, chunk, 0)
    pltpu.sync_copy(degv, out_hbm.at[wid])


@functools.cache
def _build_spmm_kernel():
    mesh = plsc.VectorSubcoreMesh(
        core_axis_name="c", subcore_axis_name="s",
        num_cores=NC, num_subcores=NS)
    return functools.partial(
        pl.kernel,
        out_type=jax.ShapeDtypeStruct((NC, NPAD, D), jnp.float32),
        mesh=mesh,
        scratch_types=[
            pltpu.VMEM((2, CHB, K), jnp.int32),
            pltpu.VMEM((2, CHB, K), jnp.int32),
            pltpu.VMEM((NBUF, K, D), jnp.float32),
            pltpu.VMEM_SHARED((NPAD, D), jnp.float32),
            pltpu.SemaphoreType.DMA,
            pltpu.SemaphoreType.DMA,
            pltpu.SemaphoreType.DMA,
            pltpu.SemaphoreType.DMA,
        ],
        compiler_params=pltpu.CompilerParams(needs_layout_passes=False),
    )(_spmm_body)


def _spmm_body(h_hbm, col_hbm, row_hbm, out_hbm, cstage, rstage, buf, aggs,
               semc, semr, semg, sems):
    c = lax.axis_index("c")
    s = lax.axis_index("s")
    wid = c * NS + s

    zeros16 = jnp.zeros((L,), jnp.float32)

    def zbody(r, carry):
        for nb in range(NBUF):
            for l in range(D // L):
                buf[nb, r, pl.ds(l * L, L)] = zeros16
        return carry

    lax.fori_loop(0, K, zbody, 0, unroll=2)
    for j in range(RPT // K):
        pltpu.sync_copy(buf.at[0], aggs.at[pl.ds(s * RPT + j * K, K)])
    plsc.subcore_barrier()

    pltpu.async_copy(
        col_hbm.at[wid, pl.ds(0, CHB)], cstage.at[0], semc).wait()
    pltpu.async_copy(
        row_hbm.at[wid, pl.ds(0, CHB)], rstage.at[0], semr).wait()
    for i in range(LA):
        pltpu.async_copy(
            buf.at[LA + i], aggs.at[rstage.at[0, 0]], sems, add=True)
        pltpu.async_copy(h_hbm.at[cstage.at[0, i]], buf.at[i], semg)

    def chunk_body(q, carry):
        a = lax.rem(q, 2)
        na = 1 - a
        nxt = (q + 1) * CHB
        pltpu.async_copy(
            col_hbm.at[wid, pl.ds(nxt, CHB)], cstage.at[na], semc)
        pltpu.async_copy(
            row_hbm.at[wid, pl.ds(nxt, CHB)], rstage.at[na], semr)
        for b in range(CHB):
            sl = b % NBUF
            pltpu.make_async_copy(
                h_hbm.at[cstage.at[a, b]], buf.at[sl], semg).wait()
            pltpu.async_copy(
                buf.at[sl], aggs.at[rstage.at[a, b]], sems, add=True)
            pltpu.make_async_copy(
                buf.at[sl], aggs.at[rstage.at[a, b]], sems).wait()
            nb = b + LA
            if nb < CHB:
                pltpu.async_copy(
                    h_hbm.at[cstage.at[a, nb]], buf.at[nb % NBUF], semg)
            else:
                if nb == CHB:
                    pltpu.make_async_copy(
                        col_hbm.at[wid, pl.ds(nxt, CHB)], cstage.at[na],
                        semc).wait()
                    pltpu.make_async_copy(
                        row_hbm.at[wid, pl.ds(nxt, CHB)], rstage.at[na],
                        semr).wait()
                pltpu.async_copy(
                    h_hbm.at[cstage.at[na, nb - CHB]], buf.at[nb % NBUF],
                    semg)
        return carry

    lax.fori_loop(0, NCH, chunk_body, 0)
    for i in range(LA):
        pltpu.make_async_copy(
            h_hbm.at[cstage.at[0, 0]], buf.at[i], semg).wait()
        pltpu.make_async_copy(
            buf.at[i], aggs.at[rstage.at[0, 0]], sems).wait()

    plsc.subcore_barrier()
    pltpu.sync_copy(aggs.at[pl.ds(s * RPT, RPT)],
                    out_hbm.at[c, pl.ds(s * RPT, RPT)])


_BN = 2048
_BG = _BN // 128


def _dinv_rows(degp):
    d = jnp.sum(degp, axis=0)
    dinv = jnp.where(d > 0, lax.rsqrt(d), 0.0)
    eye = (lax.broadcasted_iota(jnp.int32, (128, 128), 0)
           == lax.broadcasted_iota(jnp.int32, (128, 128), 1))
    ones = jnp.ones((128, D), jnp.float32)
    groups = [
        jnp.dot(jnp.where(eye, dinv[g][None, :], 0.0), ones,
                preferred_element_type=jnp.float32)
        for g in range(_BG)
    ]
    return jnp.concatenate(groups, axis=0)


def _l1_body(x_ref, degp_ref, w1t_ref, wr1t_ref, b1_ref, br1_ref,
             h_ref, r_ref):
    dinv_b = _dinv_rows(degp_ref[...])
    x = x_ref[...]
    t = jnp.dot(x, w1t_ref[...], preferred_element_type=jnp.float32) + b1_ref[...]
    h_ref[...] = dinv_b * t
    r_ref[...] = jnp.dot(x, wr1t_ref[...],
                         preferred_element_type=jnp.float32) + br1_ref[...]


def _mid_body(aggp_ref, degp_ref, r1_ref, w2t_ref, wr2t_ref, b2_ref, br2_ref,
              x1_ref, h2_ref, r2_ref):
    dinv_b = _dinv_rows(degp_ref[...])
    a = aggp_ref[0] + aggp_ref[1]
    x1 = jnp.maximum(dinv_b * a + r1_ref[...], 0.0)
    x1_ref[...] = x1
    t = jnp.dot(x1, w2t_ref[...], preferred_element_type=jnp.float32) + b2_ref[...]
    h2_ref[...] = dinv_b * t
    r2_ref[...] = jnp.dot(x1, wr2t_ref[...],
                          preferred_element_type=jnp.float32) + br2_ref[...]


def _fin_body(aggp_ref, degp_ref, r2_ref, x1_ref, wlt_ref, bl_ref, out_ref):
    dinv_b = _dinv_rows(degp_ref[...])
    a = aggp_ref[0] + aggp_ref[1]
    x2 = jnp.maximum(dinv_b * a + r2_ref[...], 0.0)
    xc = jnp.concatenate([x1_ref[...], x2], axis=1)
    t = jnp.dot(xc, wlt_ref[...], preferred_element_type=jnp.float32) + bl_ref[...]
    m = jnp.max(t, axis=1, keepdims=True)
    lse = jnp.log(jnp.sum(jnp.exp(t - m), axis=1, keepdims=True))
    out_ref[...] = t - m - lse


def _row_spec(width):
    return pl.BlockSpec((_BN, width), lambda i: (i, 0))


def _degp_spec():
    return pl.BlockSpec((NW, _BG, 128), lambda i: (0, i, 0))


def _full_spec(shape):
    nd = len(shape)
    return pl.BlockSpec(shape, lambda i, _n=nd: (0,) * _n)


_l1_call = pl.pallas_call(
    _l1_body,
    grid=(NPAD // _BN,),
    in_specs=[
        _row_spec(D),
        _degp_spec(),
        _full_spec((D, D)), _full_spec((D, D)),
        _full_spec((1, D)), _full_spec((1, D)),
    ],
    out_specs=[_row_spec(D), _row_spec(D)],
    out_shape=[jax.ShapeDtypeStruct((NPAD, D), jnp.float32)] * 2,
)

_mid_call = pl.pallas_call(
    _mid_body,
    grid=(NPAD // _BN,),
    in_specs=[
        pl.BlockSpec((NC, _BN, D), lambda i: (0, i, 0)),
        _degp_spec(), _row_spec(D),
        _full_spec((D, D)), _full_spec((D, D)),
        _full_spec((1, D)), _full_spec((1, D)),
    ],
    out_specs=[_row_spec(D), _row_spec(D), _row_spec(D)],
    out_shape=[jax.ShapeDtypeStruct((NPAD, D), jnp.float32)] * 3,
)

_fin_call = pl.pallas_call(
    _fin_body,
    grid=(NPAD // _BN,),
    in_specs=[
        pl.BlockSpec((NC, _BN, D), lambda i: (0, i, 0)),
        _degp_spec(), _row_spec(D), _row_spec(D),
        _full_spec((2 * D, C)), _full_spec((1, C)),
    ],
    out_specs=pl.BlockSpec((_BN, C), lambda i: (i, 0)),
    out_shape=jax.ShapeDtypeStruct((NPAD, C), jnp.float32),
)


def kernel(x0, edge_index, W1, b1, Wr1, br1, W2, b2, Wr2, br2, Wl, bl):
    row = edge_index[0]
    col = edge_index[1]
    npads = EPAD - E
    pad_ids = N + lax.iota(jnp.int32, npads) % NDUM
    rowp = jnp.concatenate([row, pad_ids])
    colp = jnp.concatenate([col, pad_ids])
    dumd = (N + lax.iota(jnp.int32, NW * CHD) % NDUM).reshape(NW, CHD)
    row_flat = jnp.concatenate(
        [rowp.reshape(NW, EPT), dumd], axis=1).reshape(NW, EPT // 128 + CHDR,
                                                       128)
    dumb = (N + lax.iota(jnp.int32, NW * CHB * K) % NDUM).reshape(NW, CHB, K)
    row3 = jnp.concatenate([rowp.reshape(NW, BLK, K), dumb], axis=1)
    col3 = jnp.concatenate([colp.reshape(NW, BLK, K), dumb], axis=1)

    x0p = jnp.pad(x0, ((0, NPAD - N), (0, 0)))
    b1r = b1.reshape(1, D)
    br1r = br1.reshape(1, D)
    b2r = b2.reshape(1, D)
    br2r = br2.reshape(1, D)
    blr = bl.reshape(1, C)

    degp = _deg_kernel(row_flat)
    h1p, r1p = _l1_call(x0p, degp, W1.T, Wr1.T, b1r, br1r)
    agg1 = _spmm_kernel(h1p, col3, row3)
    x1, h2p, r2p = _mid_call(agg1, degp, r1p, W2.T, Wr2.T, b2r, br2r)
    agg2 = _spmm_kernel(h2p, col3, row3)
    return _fin_call(agg2, degp, r2p, x1, Wl.T, blr)[:N]


def _deg_kernel(rows_hbm):
    return _build_deg_kernel()(rows_hbm)


def _spmm_kernel(h, col3, row3):
    return _build_spmm_kernel()(h, col3, row3)


# speedup vs baseline: 30.2325x; 1.1042x over previous
import functools

import jax
import jax.numpy as jnp
from jax import lax
from jax.experimental import pallas as pl
from jax.experimental.pallas import tpu as pltpu
from jax.experimental.pallas import tpu_sc as plsc

N = 10000
D = 128
C = 64
NPAD = 10240
NDUM = NPAD - N
NC = 2
NS = 16
NW = NC * NS
L = 16
E = 320000
K = 64
BLK = 160
EPT = BLK * K
EPAD = NW * EPT
RPT = NPAD // NS
CHB = 8
NCH = BLK // CHB
NBUF = 5
LAG = 3
LAS = 2
CHD = 2048
NCHD = EPT // CHD
CHDR = CHD // 128
NR = NPAD // 128

@functools.cache
def _build_deg_kernel():
    mesh = plsc.VectorSubcoreMesh(
        core_axis_name="c", subcore_axis_name="s",
        num_cores=NC, num_subcores=NS)
    return functools.partial(
        pl.kernel,
        out_type=jax.ShapeDtypeStruct((NW, NR, 128), jnp.float32),
        mesh=mesh,
        scratch_types=[
            pltpu.VMEM((2, CHDR, 128), jnp.int32),
            pltpu.VMEM((NR, 128), jnp.float32),
            pltpu.SemaphoreType.DMA,
        ],
        compiler_params=pltpu.CompilerParams(needs_layout_passes=False),
    )(_deg_body)


def _deg_body(rows_hbm, out_hbm, stage, degv, semd):
    c = lax.axis_index("c")
    s = lax.axis_index("s")
    wid = c * NS + s
    zeros16 = jnp.zeros((L,), jnp.float32)
    ones16 = jnp.ones((L,), jnp.float32)

    def zbody(r, carry):
        for l in range(128 // L):
            degv[r, pl.ds(l * L, L)] = zeros16
        return carry

    lax.fori_loop(0, NR, zbody, 0, unroll=2)
    pltpu.async_copy(
        rows_hbm.at[wid, pl.ds(0, CHDR)], stage.at[0], semd).wait()

    def chunk(q, carry):
        a = lax.rem(q, 2)
        na = 1 - a
        nxt = (q + 1) * CHDR
        pltpu.async_copy(rows_hbm.at[wid, pl.ds(nxt, CHDR)], stage.at[na], semd)

        def body(i, carry2):
            r = lax.shift_right_logical(i, 3)
            l = lax.rem(i, 8)
            idx = stage[a, r, pl.ds(l * L, L)]
            hi = lax.shift_right_logical(idx, 7)
            lo = lax.rem(idx, 128)
            plsc.addupdate_scatter(degv, [hi, lo], ones16)
            return carry2

        lax.fori_loop(0, CHD // L, body, 0, unroll=4)
        pltpu.make_async_copy(
            rows_hbm.at[wid, pl.ds(nxt, CHDR)], stage.at[na], semd).wait()
        return carry

    lax.fori_loop(0, NCHD, chunk, 0)
    pltpu.sync_copy(degv, out_hbm.at[wid])


@functools.cache
def _build_spmm_kernel():
    mesh = plsc.VectorSubcoreMesh(
        core_axis_name="c", subcore_axis_name="s",
        num_cores=NC, num_subcores=NS)
    return functools.partial(
        pl.kernel,
        out_type=jax.ShapeDtypeStruct((NC, NPAD, D), jnp.float32),
        mesh=mesh,
        scratch_types=[
            pltpu.VMEM((2, CHB, K), jnp.int32),
            pltpu.VMEM((2, CHB, K), jnp.int32),
            pltpu.VMEM((NBUF, K, D), jnp.float32),
            pltpu.VMEM_SHARED((NPAD, D), jnp.float32),
            pltpu.SemaphoreType.DMA,
            pltpu.SemaphoreType.DMA,
            pltpu.SemaphoreType.DMA,
            pltpu.SemaphoreType.DMA,
        ],
        compiler_params=pltpu.CompilerParams(needs_layout_passes=False),
    )(_spmm_body)


def _spmm_body(h_hbm, col_hbm, row_hbm, out_hbm, cstage, rstage, buf, aggs,
               semc, semr, semg, sems):
    c = lax.axis_index("c")
    s = lax.axis_index("s")
    wid = c * NS + s

    zeros16 = jnp.zeros((L,), jnp.float32)

    def zbody(r, carry):
        for nb in range(NBUF):
            for l in range(D // L):
                buf[nb, r, pl.ds(l * L, L)] = zeros16
        return carry

    lax.fori_loop(0, K, zbody, 0, unroll=2)
    for j in range(RPT // K):
        pltpu.sync_copy(buf.at[0], aggs.at[pl.ds(s * RPT + j * K, K)])
    plsc.subcore_barrier()

    pltpu.async_copy(
        col_hbm.at[wid, pl.ds(0, CHB)], cstage.at[0], semc).wait()
    pltpu.async_copy(
        row_hbm.at[wid, pl.ds(0, CHB)], rstage.at[0], semr).wait()
    for i in range(LAS):
        pltpu.async_copy(
            buf.at[NBUF - LAS + i], aggs.at[rstage.at[0, 0]], sems, add=True)
    for i in range(LAG):
        pltpu.async_copy(h_hbm.at[cstage.at[0, i]], buf.at[i], semg)

    def chunk_body(q, carry):
        a = lax.rem(q, 2)
        na = 1 - a
        nxt = (q + 1) * CHB
        g0 = q * CHB
        pltpu.async_copy(
            col_hbm.at[wid, pl.ds(nxt, CHB)], cstage.at[na], semc)
        pltpu.async_copy(
            row_hbm.at[wid, pl.ds(nxt, CHB)], rstage.at[na], semr)
        for b in range(CHB):
            sl = lax.rem(g0 + b, NBUF)
            sl3 = lax.rem(g0 + b + LAG, NBUF)
            pltpu.make_async_copy(
                h_hbm.at[cstage.at[a, b]], buf.at[sl], semg).wait()
            pltpu.async_copy(
                buf.at[sl], aggs.at[rstage.at[a, b]], sems, add=True)
            pltpu.make_async_copy(
                buf.at[sl], aggs.at[rstage.at[a, b]], sems).wait()
            nb = b + LAG
            if nb < CHB:
                pltpu.async_copy(
                    h_hbm.at[cstage.at[a, nb]], buf.at[sl3], semg)
            else:
                if nb == CHB:
                    pltpu.make_async_copy(
                        col_hbm.at[wid, pl.ds(nxt, CHB)], cstage.at[na],
                        semc).wait()
                    pltpu.make_async_copy(
                        row_hbm.at[wid, pl.ds(nxt, CHB)], rstage.at[na],
                        semr).wait()
                pltpu.async_copy(
                    h_hbm.at[cstage.at[na, nb - CHB]], buf.at[sl3], semg)
        return carry

    lax.fori_loop(0, NCH, chunk_body, 0)
    for i in range(LAG):
        pltpu.make_async_copy(
            h_hbm.at[cstage.at[0, 0]], buf.at[0], semg).wait()
    for i in range(LAS):
        pltpu.make_async_copy(
            buf.at[0], aggs.at[rstage.at[0, 0]], sems).wait()

    plsc.subcore_barrier()
    pltpu.sync_copy(aggs.at[pl.ds(s * RPT, RPT)],
                    out_hbm.at[c, pl.ds(s * RPT, RPT)])


_BN = 2048
_BG = _BN // 128


def _dinv_rows(degp):
    d = jnp.sum(degp, axis=0)
    dinv = jnp.where(d > 0, lax.rsqrt(d), 0.0)
    eye = (lax.broadcasted_iota(jnp.int32, (128, 128), 0)
           == lax.broadcasted_iota(jnp.int32, (128, 128), 1))
    ones = jnp.ones((128, D), jnp.float32)
    groups = [
        jnp.dot(jnp.where(eye, dinv[g][None, :], 0.0), ones,
                preferred_element_type=jnp.float32)
        for g in range(_BG)
    ]
    return jnp.concatenate(groups, axis=0)


def _l1_body(x_ref, degp_ref, w1t_ref, wr1t_ref, b1_ref, br1_ref,
             h_ref, r_ref):
    dinv_b = _dinv_rows(degp_ref[...])
    x = x_ref[...]
    t = jnp.dot(x, w1t_ref[...], preferred_element_type=jnp.float32) + b1_ref[...]
    h_ref[...] = dinv_b * t
    r_ref[...] = jnp.dot(x, wr1t_ref[...],
                         preferred_element_type=jnp.float32) + br1_ref[...]


def _mid_body(aggp_ref, degp_ref, r1_ref, w2t_ref, wr2t_ref, b2_ref, br2_ref,
              x1_ref, h2_ref, r2_ref):
    dinv_b = _dinv_rows(degp_ref[...])
    a = aggp_ref[0] + aggp_ref[1]
    x1 = jnp.maximum(dinv_b * a + r1_ref[...], 0.0)
    x1_ref[...] = x1
    t = jnp.dot(x1, w2t_ref[...], preferred_element_type=jnp.float32) + b2_ref[...]
    h2_ref[...] = dinv_b * t
    r2_ref[...] = jnp.dot(x1, wr2t_ref[...],
                          preferred_element_type=jnp.float32) + br2_ref[...]


def _fin_body(aggp_ref, degp_ref, r2_ref, x1_ref, wlt_ref, bl_ref, out_ref):
    dinv_b = _dinv_rows(degp_ref[...])
    a = aggp_ref[0] + aggp_ref[1]
    x2 = jnp.maximum(dinv_b * a + r2_ref[...], 0.0)
    xc = jnp.concatenate([x1_ref[...], x2], axis=1)
    t = jnp.dot(xc, wlt_ref[...], preferred_element_type=jnp.float32) + bl_ref[...]
    m = jnp.max(t, axis=1, keepdims=True)
    lse = jnp.log(jnp.sum(jnp.exp(t - m), axis=1, keepdims=True))
    out_ref[...] = t - m - lse


def _row_spec(width):
    return pl.BlockSpec((_BN, width), lambda i: (i, 0))


def _degp_spec():
    return pl.BlockSpec((NW, _BG, 128), lambda i: (0, i, 0))


def _full_spec(shape):
    nd = len(shape)
    return pl.BlockSpec(shape, lambda i, _n=nd: (0,) * _n)


_l1_call = pl.pallas_call(
    _l1_body,
    grid=(NPAD // _BN,),
    in_specs=[
        _row_spec(D),
        _degp_spec(),
        _full_spec((D, D)), _full_spec((D, D)),
        _full_spec((1, D)), _full_spec((1, D)),
    ],
    out_specs=[_row_spec(D), _row_spec(D)],
    out_shape=[jax.ShapeDtypeStruct((NPAD, D), jnp.float32)] * 2,
)

_mid_call = pl.pallas_call(
    _mid_body,
    grid=(NPAD // _BN,),
    in_specs=[
        pl.BlockSpec((NC, _BN, D), lambda i: (0, i, 0)),
        _degp_spec(), _row_spec(D),
        _full_spec((D, D)), _full_spec((D, D)),
        _full_spec((1, D)), _full_spec((1, D)),
    ],
    out_specs=[_row_spec(D), _row_spec(D), _row_spec(D)],
    out_shape=[jax.ShapeDtypeStruct((NPAD, D), jnp.float32)] * 3,
)

_fin_call = pl.pallas_call(
    _fin_body,
    grid=(NPAD // _BN,),
    in_specs=[
        pl.BlockSpec((NC, _BN, D), lambda i: (0, i, 0)),
        _degp_spec(), _row_spec(D), _row_spec(D),
        _full_spec((2 * D, C)), _full_spec((1, C)),
    ],
    out_specs=pl.BlockSpec((_BN, C), lambda i: (i, 0)),
    out_shape=jax.ShapeDtypeStruct((N, C), jnp.float32),
)


def kernel(x0, edge_index, W1, b1, Wr1, br1, W2, b2, Wr2, br2, Wl, bl):
    row = edge_index[0]
    col = edge_index[1]
    npads = EPAD - E
    pad_ids = N + lax.iota(jnp.int32, npads) % NDUM
    rowp = jnp.concatenate([row, pad_ids])
    colp = jnp.concatenate([col, pad_ids])
    dumd = (N + lax.iota(jnp.int32, NW * CHD) % NDUM).reshape(NW, CHD)
    row_flat = jnp.concatenate(
        [rowp.reshape(NW, EPT), dumd], axis=1).reshape(NW, EPT // 128 + CHDR,
                                                       128)
    dumb = (N + lax.iota(jnp.int32, NW * CHB * K) % NDUM).reshape(NW, CHB, K)
    row3 = jnp.concatenate([rowp.reshape(NW, BLK, K), dumb], axis=1)
    col3 = jnp.concatenate([colp.reshape(NW, BLK, K), dumb], axis=1)

    x0p = jnp.pad(x0, ((0, NPAD - N), (0, 0)))
    b1r = b1.reshape(1, D)
    br1r = br1.reshape(1, D)
    b2r = b2.reshape(1, D)
    br2r = br2.reshape(1, D)
    blr = bl.reshape(1, C)

    degp = _deg_kernel(row_flat)
    h1p, r1p = _l1_call(x0p, degp, W1.T, Wr1.T, b1r, br1r)
    agg1 = _spmm_kernel(h1p, col3, row3)
    x1, h2p, r2p = _mid_call(agg1, degp, r1p, W2.T, Wr2.T, b2r, br2r)
    agg2 = _spmm_kernel(h2p, col3, row3)
    return _fin_call(agg2, degp, r2p, x1, Wl.T, blr)


def _deg_kernel(rows_hbm):
    return _build_deg_kernel()(rows_hbm)


def _spmm_kernel(h, col3, row3):
    return _build_spmm_kernel()(h, col3, row3)
